# baseline (speedup 1.0000x reference)
import jax
import jax.numpy as jnp
from jax import lax
from jax.experimental import pallas as pl
from jax.experimental.pallas import tpu as pltpu

B, H, W = 8192, 64, 64
HW = H * W
BLOCK_LANES = 1024
NUM_BLOCKS = B // BLOCK_LANES


def _loss_block_kernel(x_ref, out_ref):
    xb = x_ref[...]
    x3 = xb.reshape(H, W, -1)
    m = jnp.max(xb, axis=0, keepdims=True)
    kf3 = lax.broadcasted_iota(jnp.int32, (1, W, 1), 1).astype(jnp.float32)
    firstk = jnp.min(jnp.where(x3 == m[None], kf3, float(W)),
                     axis=1)
    jcf = lax.broadcasted_iota(jnp.int32, (H, 1), 0).astype(jnp.float32)
    idx = jnp.min(jnp.where(firstk < W, W * jcf + firstk, float(HW)),
                  axis=0, keepdims=True)
    mx = jnp.floor(idx * (1.0 / W))
    my = idx - W * mx
    g = jnp.sum(x3, axis=1)
    gk = jnp.sum(x3, axis=0)
    jc = jcf
    s0 = jnp.sum(g, axis=0, keepdims=True)
    sj = jnp.sum(g * jc, axis=0, keepdims=True)
    sj2 = jnp.sum(g * (jc * jc), axis=0, keepdims=True)
    sk = jnp.sum(gk * jc, axis=0, keepdims=True)
    sk2 = jnp.sum(gk * (jc * jc), axis=0, keepdims=True)
    loss_l = ((mx * mx + my * my) * s0 - 2.0 * (mx * sj + my * sk)
              + sj2 + sk2)
    out_ref[...] = jnp.full((1, 1, 128), jnp.sum(loss_l), dtype=jnp.float32)


def kernel(x):
    xt = x.transpose(1, 2, 0).reshape(HW, B)
    partials = pl.pallas_call(
        _loss_block_kernel,
        grid=(NUM_BLOCKS,),
        in_specs=[
            pl.BlockSpec((HW, BLOCK_LANES), lambda i: (0, i)),
        ],
        out_specs=pl.BlockSpec((1, 1, 128), lambda i: (i, 0, 0)),
        out_shape=jax.ShapeDtypeStruct((NUM_BLOCKS, 1, 128), jnp.float32),
        compiler_params=pltpu.CompilerParams(
            dimension_semantics=("parallel",),
        ),
    )(xt)
    return jnp.sum(partials[:, 0, 0]).reshape(1)


# speedup vs baseline: 1.1259x; 1.1259x over previous
import jax
import jax.numpy as jnp
from jax import lax
from jax.experimental import pallas as pl
from jax.experimental.pallas import tpu as pltpu

B, H, W = 8192, 64, 64
HW = H * W
BLOCK_LANES = 1024
NUM_BLOCKS = B // BLOCK_LANES


def _loss_block_kernel(x_ref, out_ref):
    xb = x_ref[...]
    x3 = xb.reshape(H, W, -1)
    m = jnp.max(xb, axis=0, keepdims=True)
    r = lax.broadcasted_iota(jnp.int32, (HW, 1), 0)
    rf = r.astype(jnp.float32)
    idx = jnp.min(jnp.where(xb == m, rf, float(HW)), axis=0,
                  keepdims=True)
    mx = jnp.floor(idx * (1.0 / W))
    my = idx - W * mx
    g = jnp.sum(x3, axis=1)
    gk = jnp.sum(x3, axis=0)
    jc = lax.broadcasted_iota(jnp.int32, (H, 1), 0).astype(jnp.float32)
    s0 = jnp.sum(g, axis=0, keepdims=True)
    sj = jnp.sum(g * jc, axis=0, keepdims=True)
    sj2 = jnp.sum(g * (jc * jc), axis=0, keepdims=True)
    sk = jnp.sum(gk * jc, axis=0, keepdims=True)
    sk2 = jnp.sum(gk * (jc * jc), axis=0, keepdims=True)
    loss_l = ((mx * mx + my * my) * s0 - 2.0 * (mx * sj + my * sk)
              + sj2 + sk2)
    out_ref[...] = jnp.full((1, 1, 128), jnp.sum(loss_l), dtype=jnp.float32)


def kernel(x):
    xt = x.transpose(1, 2, 0).reshape(HW, B)
    partials = pl.pallas_call(
        _loss_block_kernel,
        grid=(NUM_BLOCKS,),
        in_specs=[
            pl.BlockSpec((HW, BLOCK_LANES), lambda i: (0, i)),
        ],
        out_specs=pl.BlockSpec((1, 1, 128), lambda i: (i, 0, 0)),
        out_shape=jax.ShapeDtypeStruct((NUM_BLOCKS, 1, 128), jnp.float32),
        compiler_params=pltpu.CompilerParams(
            dimension_semantics=("parallel",),
        ),
    )(xt)
    return jnp.sum(partials[:, 0, 0]).reshape(1)
